# Initial kernel scaffold; baseline (speedup 1.0000x reference)
import jax
import jax.numpy as jnp
from jax.experimental import pallas as pl


def kernel(x, edge_index, Wg0_0, Wg0_1, Wg1_0, Wg1_1, p0, p1):
    raise NotImplementedError("write your pallas kernel here")


# speedup vs baseline: 1.0000x; 1.0000x over previous
import jax
jax.config.update("jax_enable_x64", True)
import jax.numpy as jnp
from jax.experimental import pallas as pl

EGO_RANGE = 2


def _gcn_conv(x, src, dst, W, n):
    h = x @ W
    ones = jnp.ones(src.shape[0], dtype=x.dtype)
    deg = jax.ops.segment_sum(ones, dst, num_segments=n) + 1.0
    norm = jax.lax.rsqrt(deg)
    coef = norm[src] * norm[dst]
    agg = jax.ops.segment_sum(h[src] * coef[:, None], dst, num_segments=n)
    return agg + h * (norm * norm)[:, None]


def _pool(x, edge_index, p, n):
    src, dst = edge_index[0], edge_index[1]
    scores = jnp.tanh(x @ p)
    ones = jnp.ones(src.shape[0], dtype=x.dtype)
    deg = jnp.maximum(jax.ops.segment_sum(ones, dst, num_segments=n), 1.0)
    for _ in range(EGO_RANGE):
        scores = 0.5 * scores + 0.5 * jax.ops.segment_sum(scores[src], dst, num_segments=n) / deg
    k = n // 2
    vals, idx = jax.lax.top_k(scores, k)
    x_new = x[idx] * vals[:, None]
    pos = jnp.full((n,), -1, dtype=src.dtype).at[idx].set(jnp.arange(k, dtype=src.dtype))
    s2 = pos[src]
    d2 = pos[dst]
    valid = (s2 >= 0) & (d2 >= 0)
    new_ei = jnp.where(valid[None, :], jnp.stack([s2, d2]), 0)
    return x_new, new_ei, idx, vals


def kernel(x, edge_index, Wg0_0, Wg0_1, Wg1_0, Wg1_1, p0, p1):
    Ws = [[Wg0_0, Wg0_1], [Wg1_0, Wg1_1]]
    ps = [p0, p1]
    x_list = []
    ei_list = [edge_index]
    S_list = []
    ei = edge_index
    n = x.shape[0]
    for i in range(2):
        src, dst = ei[0], ei[1]
        for j, W in enumerate(Ws[i]):
            x = _gcn_conv(x, src, dst, W, n)
            if j == 0:
                x = jax.nn.relu(x)
        x = jax.nn.relu(x)
        x_list.append(x)
        x, ei, S, _ = _pool(x, ei, ps[i], n)
        n = x.shape[0]
        S_list.append(S)
        ei_list.append(ei)
    return (x, x_list[0], x_list[1], ei_list[0], ei_list[1], ei_list[2], S_list[0], S_list[1])


# speedup vs baseline: 1.8407x; 1.8407x over previous
import functools

import jax
jax.config.update("jax_enable_x64", True)
import jax.numpy as jnp
from jax import lax
from jax.experimental import pallas as pl
from jax.experimental.pallas import tpu as pltpu
from jax.experimental.pallas import tpu_sc as plsc

F = 128
E = 320000
NC, NS = 2, 16
NW = NC * NS
R = 80
EPADR = NW * R
EPAD = EPADR * 128

_i32 = jnp.int32


def _mesh():
    return plsc.VectorSubcoreMesh(core_axis_name="c", subcore_axis_name="s")


def _im(*vals):
    def f(i):
        i = i.astype(jnp.int32) if hasattr(i, "astype") else _i32(i)
        z = _i32(0)
        return tuple(i if v == "i" else z for v in vals)
    return f


@functools.partial(jax.jit, static_argnames=("npad",))
def _sc_deg(dstp, *, npad):
    @functools.partial(
        pl.kernel,
        out_type=jax.ShapeDtypeStruct((NW * npad,), jnp.float32),
        mesh=_mesh(),
        compiler_params=pltpu.CompilerParams(needs_layout_passes=False),
        scratch_types=[
            pltpu.VMEM((R, 128), jnp.int32),
            pltpu.VMEM((npad,), jnp.float32),
        ],
    )
    def k(dst_hbm, out_hbm, dst_v, acc_v):
        wid = lax.axis_index("s") * NC + lax.axis_index("c")
        pltpu.sync_copy(dst_hbm.at[pl.ds(wid * R, R), :], dst_v)

        @pl.loop(_i32(0), _i32(npad // 16), step=_i32(1))
        def _(i):
            acc_v[pl.ds(i * 16, 16)] = jnp.zeros((16,), jnp.float32)

        ones = jnp.full((16,), 1.0, jnp.float32)

        @pl.loop(_i32(0), _i32(R), step=_i32(1))
        def _(j):
            for s in range(8):
                d = dst_v[j, pl.ds(s * 16, 16)]
                plsc.addupdate_scatter(acc_v, [d], ones)

        pltpu.sync_copy(acc_v, out_hbm.at[pl.ds(wid * npad, npad)])

    return k(dstp)


@functools.partial(jax.jit, static_argnames=("kpad",))
def _sc_gather_rows(xp, idxp, *, kpad):
    b = kpad // NW
    nch = b // 80
    ch = b // nch

    @functools.partial(
        pl.kernel,
        out_type=jax.ShapeDtypeStruct((kpad, F), jnp.float32),
        mesh=_mesh(),
        compiler_params=pltpu.CompilerParams(needs_layout_passes=False),
        scratch_types=[
            pltpu.VMEM((ch,), jnp.int32),
            pltpu.VMEM((ch, F), jnp.float32),
            pltpu.SemaphoreType.DMA,
        ],
    )
    def k(x_hbm, idx_hbm, out_hbm, idx_v, rows_v, sem):
        wid = lax.axis_index("s") * NC + lax.axis_index("c")
        for c in range(nch):
            base = wid * b + c * ch
            pltpu.sync_copy(idx_hbm.at[pl.ds(base, ch)], idx_v)
            pltpu.async_copy(x_hbm.at[idx_v], rows_v, sem).wait()
            pltpu.sync_copy(rows_v, out_hbm.at[pl.ds(base, ch), :])

    return k(xp, idxp)


@functools.partial(jax.jit, static_argnames=("npad", "kk", "kpad", "sent"))
def _sc_remap(srcp, dstp, idxp, *, npad, kk, kpad, sent):
    @functools.partial(
        pl.kernel,
        out_type=jax.ShapeDtypeStruct((2, EPADR, 128), jnp.int32),
        mesh=_mesh(),
        compiler_params=pltpu.CompilerParams(needs_layout_passes=False),
        scratch_types=[
            pltpu.VMEM((R, 128), jnp.int32),
            pltpu.VMEM((R, 128), jnp.int32),
            pltpu.VMEM((npad,), jnp.int32),
            pltpu.VMEM((kpad,), jnp.int32),
            pltpu.VMEM((R, 128), jnp.int32),
            pltpu.VMEM((R, 128), jnp.int32),
        ],
    )
    def k(src_hbm, dst_hbm, idx_hbm, out_hbm,
          src_v, dst_v, pos_v, idx_v, os_v, od_v):
        wid = lax.axis_index("s") * NC + lax.axis_index("c")
        pltpu.sync_copy(src_hbm.at[pl.ds(wid * R, R), :], src_v)
        pltpu.sync_copy(dst_hbm.at[pl.ds(wid * R, R), :], dst_v)
        pltpu.sync_copy(idx_hbm, idx_v)

        @pl.loop(_i32(0), _i32(npad // 16), step=_i32(1))
        def _(i):
            pos_v[pl.ds(i * 16, 16)] = jnp.full((16,), -1, jnp.int32)

        iota = lax.iota(jnp.int32, 16)

        @pl.loop(_i32(0), _i32(kpad // 16), step=_i32(1))
        def _(i):
            c16 = idx_v[pl.ds(i * 16, 16)]
            posv = i * 16 + iota
            plsc.store_scatter(pos_v, [c16], posv, mask=posv < kk)

        @pl.loop(_i32(0), _i32(R), step=_i32(1))
        def _(j):
            for s in range(8):
                sc = src_v[j, pl.ds(s * 16, 16)]
                dc = dst_v[j, pl.ds(s * 16, 16)]
                s2 = plsc.load_gather(pos_v, [sc])
                d2 = plsc.load_gather(pos_v, [dc])
                pe = (wid * R + j) * 128 + s * 16 + iota
                real = pe < E
                keep = (s2 >= 0) & (d2 >= 0) & real
                os_v[j, pl.ds(s * 16, 16)] = jnp.where(keep, s2, 0)
                od_v[j, pl.ds(s * 16, 16)] = jnp.where(
                    keep, d2, jnp.where(real, _i32(0), _i32(sent)))

        pltpu.sync_copy(os_v, out_hbm.at[_i32(0), pl.ds(wid * R, R), :])
        pltpu.sync_copy(od_v, out_hbm.at[_i32(1), pl.ds(wid * R, R), :])

    return k(srcp, dstp, idxp)


def _tc_mm(xp, W, npad):
    rb = 1280
    g = npad // rb

    def body(x_ref, w_ref, o_ref):
        o_ref[...] = jnp.dot(x_ref[...], w_ref[...],
                             preferred_element_type=jnp.float32)

    return pl.pallas_call(
        body,
        grid=(g,),
        in_specs=[pl.BlockSpec((rb, F), _im("i", "z")),
                  pl.BlockSpec((F, F), _im("z", "z"))],
        out_specs=pl.BlockSpec((rb, F), _im("i", "z")),
        out_shape=jax.ShapeDtypeStruct((npad, F), jnp.float32),
    )(xp, W)


def _tc_combine(agg, h, nn_full, npad, relu):
    rb = 1280
    g = npad // rb

    def body(a_ref, h_ref, n_ref, o_ref):
        o = a_ref[...] + h_ref[...] * n_ref[...]
        o_ref[...] = jnp.maximum(o, 0.0) if relu else o

    return pl.pallas_call(
        body,
        grid=(g,),
        in_specs=[pl.BlockSpec((rb, F), _im("i", "z")),
                  pl.BlockSpec((rb, F), _im("i", "z")),
                  pl.BlockSpec((rb, F), _im("i", "z"))],
        out_specs=pl.BlockSpec((rb, F), _im("i", "z")),
        out_shape=jax.ShapeDtypeStruct((npad, F), jnp.float32),
    )(agg, h, nn_full)


def _tc_tanh_matvec(xp, p, npad):
    rb = 1280
    g = npad // rb

    def body(x_ref, p_ref, o_ref):
        s = jnp.dot(x_ref[...], p_ref[...], preferred_element_type=jnp.float32)
        o_ref[...] = jnp.tanh(s)

    return pl.pallas_call(
        body,
        grid=(g,),
        in_specs=[pl.BlockSpec((rb, F), _im("i", "z")),
                  pl.BlockSpec((F, F), _im("z", "z"))],
        out_specs=pl.BlockSpec((rb, F), _im("i", "z")),
        out_shape=jax.ShapeDtypeStruct((npad, F), jnp.float32),
    )(xp, jnp.broadcast_to(p[:, None], (F, F)))


def _tc_scale_rows(xg, vals_full, kpad):
    rb = 1280
    g = kpad // rb

    def body(x_ref, v_ref, o_ref):
        o_ref[...] = x_ref[...] * v_ref[...]

    return pl.pallas_call(
        body,
        grid=(g,),
        in_specs=[pl.BlockSpec((rb, F), _im("i", "z")),
                  pl.BlockSpec((rb, F), _im("i", "z"))],
        out_specs=pl.BlockSpec((rb, F), _im("i", "z")),
        out_shape=jax.ShapeDtypeStruct((kpad, F), jnp.float32),
    )(xg, vals_full)


def _level(xp, srcp, dstp, src, dst, Ws, p, n, npad, kpad):
    deg = _sc_deg(dstp, npad=npad).reshape(NW, npad).sum(0)[:n]
    norm = lax.rsqrt(deg + 1.0)
    nn = norm * norm
    nn_full = jnp.broadcast_to(
        jnp.concatenate([nn, jnp.zeros(npad - n, jnp.float32)])[:, None],
        (npad, F))
    coef = norm[src] * norm[dst]

    xc = xp
    for W in Ws:
        h = _tc_mm(xc, W, npad)
        agg = jax.ops.segment_sum(h[:n][src] * coef[:, None], dst,
                                  num_segments=n)
        aggp = jnp.concatenate([agg, jnp.zeros((npad - n, F), jnp.float32)])
        xc = _tc_combine(aggp, h, nn_full, npad, relu=True)

    scores = _tc_tanh_matvec(xc, p, npad)[:n, 0]
    degp = jnp.maximum(deg, 1.0)
    for _ in range(2):
        scores = 0.5 * scores + 0.5 * jax.ops.segment_sum(
            scores[src], dst, num_segments=n) / degp

    kk = n // 2
    vals, idx = lax.top_k(scores, kk)

    idxp = jnp.concatenate([idx.astype(jnp.int32),
                            jnp.zeros(kpad - kk, jnp.int32)])
    valsp = jnp.concatenate([vals, jnp.zeros(kpad - kk, jnp.float32)])
    vals_full = jnp.broadcast_to(valsp[:, None], (kpad, F))

    xg = _sc_gather_rows(xc, idxp, kpad=kpad)
    xnew = _tc_scale_rows(xg, vals_full, kpad)
    einew = _sc_remap(srcp, dstp, idxp, npad=npad, kk=kk, kpad=kpad,
                      sent=kpad - 1)
    return xc, xnew, einew, idx


def kernel(x, edge_index, Wg0_0, Wg0_1, Wg1_0, Wg1_1, p0, p1):
    ei32 = edge_index.astype(jnp.int32)
    npad0, npad1, npad2 = 10240, 5120, 2560

    src0p = jnp.concatenate(
        [ei32[0], jnp.zeros(EPAD - E, jnp.int32)]).reshape(EPADR, 128)
    dst0p = jnp.concatenate(
        [ei32[1], jnp.full(EPAD - E, npad0 - 1, jnp.int32)]).reshape(EPADR, 128)
    x0 = jnp.concatenate(
        [x.astype(jnp.float32), jnp.zeros((npad0 - x.shape[0], F), jnp.float32)])

    x2_0, xnew0, ei1p, idx0 = _level(
        x0, src0p, dst0p, edge_index[0], edge_index[1],
        [Wg0_0, Wg0_1], p0, n=10000, npad=npad0, kpad=npad1)
    ei1_flat = ei1p.reshape(2, EPAD)[:, :E]
    src1 = ei1_flat[0].astype(edge_index.dtype)
    dst1 = ei1_flat[1].astype(edge_index.dtype)
    x2_1, xnew1, ei2p, idx1 = _level(
        xnew0, ei1p[0], ei1p[1], src1, dst1,
        [Wg1_0, Wg1_1], p1, n=5000, npad=npad1, kpad=npad2)

    eidt = edge_index.dtype
    ei1 = ei1_flat.astype(eidt)
    ei2 = ei2p.reshape(2, EPAD)[:, :E].astype(eidt)
    return (xnew1[:2500], x2_0[:10000], x2_1[:5000],
            edge_index, ei1, ei2, idx0, idx1)
